# baseline (speedup 1.0000x reference)
import functools

import jax
import jax.numpy as jnp
from jax import lax
from jax.experimental import pallas as pl
from jax.experimental.pallas import tpu as pltpu
from jax.experimental.pallas import tpu_sc as plsc

N = 10000
D = 128
EMB = 64
E = 320000
NC, NS = 2, 16
NW = NC * NS
K = 128
EPW = 10240
CH = EPW // K
IH = CH // 2
EPAD = NW * EPW
TCH = EPAD // K
Q0 = 136
Q1 = 24
NPAD = 10112
TR = NPAD // NS

_mesh = plsc.VectorSubcoreMesh(core_axis_name="c", subcore_axis_name="s")


def _fill(ref, nrows, val):
    def body(i, _):
        ref[i // 8, pl.ds((i % 8) * 16, 16)] = jnp.full((16,), val, jnp.float32)
        return 0
    lax.fori_loop(0, nrows * 8, body, 0)


def _zero_slab(zsrc, acc_s, base):
    for j in range(TR // K):
        pltpu.sync_copy(zsrc, acc_s.at[pl.ds(base + j * K, K)])
    rem = TR - (TR // K) * K
    if rem:
        pltpu.sync_copy(zsrc.at[pl.ds(0, rem)],
                        acc_s.at[pl.ds(base + (TR // K) * K, rem)])


@functools.partial(
    pl.kernel,
    out_type=jax.ShapeDtypeStruct((NC, NPAD, D), jnp.float32),
    mesh=_mesh,
    scratch_types=[
        pltpu.VMEM((8, K), jnp.int32),
        pltpu.VMEM((K, D), jnp.float32),
        pltpu.VMEM((K, D), jnp.float32),
        pltpu.SemaphoreType.DMA,
        pltpu.SemaphoreType.DMA,
        pltpu.VMEM_SHARED((NPAD, D), jnp.float32),
    ],
)
def _sc_sum(x_hbm, eidx_hbm, sum_out,
            eidx, rows0, rows1, g0, g1, acc_s):
    c = lax.axis_index("c")
    s = lax.axis_index("s")
    base = s * TR
    rows = (rows0, rows1)
    gsem = (g0, g1)

    _fill(rows0, K, 0.0)
    _zero_slab(rows0, acc_s, base)

    plsc.subcore_barrier()

    def issue_gather(j, b):
        for h in range(2):
            pltpu.async_copy(x_hbm.at[eidx.at[2 * j, pl.ds(64 * h, 64)]],
                             rows[b].at[pl.ds(64 * h, 64)], gsem[b])

    def run_range(w0, nstages):
        def stage_body(g, _):
            ch0 = w0 + 4 * g
            pltpu.sync_copy(eidx_hbm.at[0, pl.ds(2 * ch0, 8)], eidx)
            issue_gather(0, 0)
            issue_gather(1, 1)
            for j in range(4):
                b = j % 2
                pltpu.make_async_copy(x_hbm.at[pl.ds(0, K)], rows[b],
                                      gsem[b]).wait()
                pltpu.sync_copy(rows[b], acc_s.at[eidx.at[2 * j + 1]], add=True)
                if j + 2 < 4:
                    issue_gather(j + 2, b)
            return 0
        lax.fori_loop(0, nstages, stage_body, 0)

    @pl.when(c == 0)
    def _():
        run_range(s * Q0, Q0 // 4)

    @pl.when(c == 1)
    def _():
        run_range(NS * Q0 + s * Q1, Q1 // 4)

    plsc.subcore_barrier()

    pltpu.sync_copy(acc_s.at[pl.ds(base, TR)], sum_out.at[c, pl.ds(base, TR)])


@functools.partial(
    pl.kernel,
    out_type=jax.ShapeDtypeStruct((NC, NPAD, D), jnp.float32),
    mesh=_mesh,
    scratch_types=[
        pltpu.VMEM((IH, K), jnp.int32),
        pltpu.VMEM((K, D), jnp.float32),
        pltpu.SemaphoreType.DMA,
        pltpu.VMEM_SHARED((NPAD, D), jnp.float32),
    ],
)
def _sc_ones(dst_hbm, cnt_out, didx, ones_v, ssem, acc_s):
    c = lax.axis_index("c")
    s = lax.axis_index("s")
    w = c * NS + s
    base = s * TR

    _fill(ones_v, K, 0.0)
    _zero_slab(ones_v, acc_s, base)
    _fill(ones_v, K, 1.0)

    plsc.subcore_barrier()

    def drain():
        pltpu.make_async_copy(ones_v, acc_s.at[pl.ds(0, K)], ssem).wait()

    def edge_body(i, _):
        pltpu.async_copy(ones_v, acc_s.at[didx.at[i]], ssem, add=True)

        @pl.when(i >= 3)
        def _():
            drain()
        return 0

    for half in range(2):
        pltpu.sync_copy(dst_hbm.at[0, pl.ds(w * CH + half * IH, IH)], didx)
        lax.fori_loop(0, IH, edge_body, 0)
        for _ in range(3):
            drain()

    plsc.subcore_barrier()

    pltpu.sync_copy(acc_s.at[pl.ds(base, TR)], cnt_out.at[c, pl.ds(base, TR)])


_BM = 400
_G = N // _BM


def _dotT(a, b):
    return lax.dot_general(a, b, (((1,), (1,)), ((), ())),
                           preferred_element_type=jnp.float32)


def _tc1_body(p_ref, c_ref, x_ref, wl_ref, wr_ref, b_ref, o_ref):
    cnt = jnp.maximum(c_ref[0, :, 0:1] + c_ref[1, :, 0:1], 1.0)
    mean = (p_ref[0] + p_ref[1]) / cnt
    h = _dotT(mean, wl_ref[...]) + _dotT(x_ref[...], wr_ref[...]) + b_ref[...]
    o_ref[...] = jnp.maximum(h, 0.0)


_tc1 = pl.pallas_call(
    _tc1_body,
    grid=(_G,),
    in_specs=[
        pl.BlockSpec((NC, _BM, D), lambda i: (0, i, 0)),
        pl.BlockSpec((NC, _BM, D), lambda i: (0, i, 0)),
        pl.BlockSpec((_BM, D), lambda i: (i, 0)),
        pl.BlockSpec((D, D), lambda i: (0, 0)),
        pl.BlockSpec((D, D), lambda i: (0, 0)),
        pl.BlockSpec((1, D), lambda i: (0, 0)),
    ],
    out_specs=pl.BlockSpec((_BM, D), lambda i: (i, 0)),
    out_shape=jax.ShapeDtypeStruct((N, D), jnp.float32),
)


def _tc2_body(q_ref, c_ref, h_ref, w2l_ref, w2r_ref, b2_ref,
              wm1_ref, bm1_ref, g2_ref, bt_ref, wm2_ref, bm2_ref, o_ref):
    cnt = jnp.maximum(c_ref[0, :, 0:1] + c_ref[1, :, 0:1], 1.0)
    mean = (q_ref[0] + q_ref[1]) / cnt
    h2 = _dotT(mean, w2l_ref[...]) + _dotT(h_ref[...], w2r_ref[...]) + b2_ref[...]
    h2 = jnp.maximum(h2, 0.0)
    z = (_dotT(h2, wm1_ref[...]) + bm1_ref[...]) * g2_ref[...] + bt_ref[...]
    z = jnp.maximum(z, 0.0)
    o_ref[...] = _dotT(z, wm2_ref[...]) + bm2_ref[...]


_tc2 = pl.pallas_call(
    _tc2_body,
    grid=(_G,),
    in_specs=[
        pl.BlockSpec((NC, _BM, D), lambda i: (0, i, 0)),
        pl.BlockSpec((NC, _BM, D), lambda i: (0, i, 0)),
        pl.BlockSpec((_BM, D), lambda i: (i, 0)),
        pl.BlockSpec((D, D), lambda i: (0, 0)),
        pl.BlockSpec((D, D), lambda i: (0, 0)),
        pl.BlockSpec((1, D), lambda i: (0, 0)),
        pl.BlockSpec((D, D), lambda i: (0, 0)),
        pl.BlockSpec((1, D), lambda i: (0, 0)),
        pl.BlockSpec((1, D), lambda i: (0, 0)),
        pl.BlockSpec((1, D), lambda i: (0, 0)),
        pl.BlockSpec((EMB, D), lambda i: (0, 0)),
        pl.BlockSpec((1, EMB), lambda i: (0, 0)),
    ],
    out_specs=pl.BlockSpec((_BM, EMB), lambda i: (i, 0)),
    out_shape=jax.ShapeDtypeStruct((N, EMB), jnp.float32),
)


def kernel(x, edge_index, W1l, b1l, W1r, W2l, b2l, W2r,
           Wm1, bm1, gamma, beta, Wm2, bm2):
    ei = edge_index.astype(jnp.int32)
    pad = EPAD - E
    src_p = jnp.concatenate([ei[0], jnp.zeros((pad,), jnp.int32)]).reshape(TCH, K)
    dst_p = jnp.concatenate([ei[1], jnp.full((pad,), N, jnp.int32)]).reshape(TCH, K)
    eidx_p = jnp.stack([src_p, dst_p], axis=1).reshape(1, 2 * TCH, K)

    c1 = _sc_ones(dst_p.reshape(1, TCH, K))
    s1 = _sc_sum(x, eidx_p)
    h = _tc1(s1, c1, x, W1l, W1r, b1l.reshape(1, D))
    s2 = _sc_sum(h, eidx_p)
    g2 = (gamma / jnp.sqrt(1.0 + 1e-5)).reshape(1, D)
    out = _tc2(s2, c1, h, W2l, W2r, b2l.reshape(1, D),
               Wm1, bm1.reshape(1, D), g2, beta.reshape(1, D),
               Wm2, bm2.reshape(1, EMB))
    return out


# speedup vs baseline: 1.0918x; 1.0918x over previous
import functools

import jax
import jax.numpy as jnp
from jax import lax
from jax.experimental import pallas as pl
from jax.experimental.pallas import tpu as pltpu
from jax.experimental.pallas import tpu_sc as plsc

N = 10000
D = 128
EMB = 64
E = 320000
NC, NS = 2, 16
NW = NC * NS
K = 128
EPW = 10240
CH = EPW // K
IH = CH // 2
EPAD = NW * EPW
TCH = EPAD // K
Q0 = 144
Q1 = 16
NPAD = 10112
TR = NPAD // NS

_mesh = plsc.VectorSubcoreMesh(core_axis_name="c", subcore_axis_name="s")


def _fill(ref, nrows, val):
    def body(i, _):
        ref[i // 8, pl.ds((i % 8) * 16, 16)] = jnp.full((16,), val, jnp.float32)
        return 0
    lax.fori_loop(0, nrows * 8, body, 0)


def _zero_slab(zsrc, acc_s, base):
    for j in range(TR // K):
        pltpu.sync_copy(zsrc, acc_s.at[pl.ds(base + j * K, K)])
    rem = TR - (TR // K) * K
    if rem:
        pltpu.sync_copy(zsrc.at[pl.ds(0, rem)],
                        acc_s.at[pl.ds(base + (TR // K) * K, rem)])


@functools.partial(
    pl.kernel,
    out_type=jax.ShapeDtypeStruct((NC, NPAD, D), jnp.float32),
    mesh=_mesh,
    scratch_types=[
        pltpu.VMEM((4, K), jnp.int32),
        pltpu.VMEM((4, K), jnp.int32),
        pltpu.VMEM((K, D), jnp.float32),
        pltpu.VMEM((K, D), jnp.float32),
        pltpu.SemaphoreType.DMA,
        pltpu.SemaphoreType.DMA,
        pltpu.VMEM_SHARED((NPAD, D), jnp.float32),
    ],
)
def _sc_sum(x_hbm, src_hbm, dst_hbm, sum_out,
            sidx, didx, rows0, rows1, g0, g1, acc_s):
    c = lax.axis_index("c")
    s = lax.axis_index("s")
    base = s * TR
    rows = (rows0, rows1)
    gsem = (g0, g1)

    _fill(rows0, K, 0.0)
    _zero_slab(rows0, acc_s, base)

    plsc.subcore_barrier()

    def issue_gather(j, b):
        for h in range(2):
            pltpu.async_copy(x_hbm.at[sidx.at[j, pl.ds(64 * h, 64)]],
                             rows[b].at[pl.ds(64 * h, 64)], gsem[b])

    def run_range(w0, nstages):
        def stage_body(g, _):
            ch0 = w0 + 4 * g
            pltpu.sync_copy(src_hbm.at[0, pl.ds(ch0, 4)], sidx)
            pltpu.sync_copy(dst_hbm.at[0, pl.ds(ch0, 4)], didx)
            issue_gather(0, 0)
            issue_gather(1, 1)
            for j in range(4):
                b = j % 2
                pltpu.make_async_copy(x_hbm.at[pl.ds(0, K)], rows[b],
                                      gsem[b]).wait()
                pltpu.sync_copy(rows[b], acc_s.at[didx.at[j]], add=True)
                if j + 2 < 4:
                    issue_gather(j + 2, b)
            return 0
        lax.fori_loop(0, nstages, stage_body, 0)

    @pl.when(c == 0)
    def _():
        run_range(s * Q0, Q0 // 4)

    @pl.when(c == 1)
    def _():
        run_range(NS * Q0 + s * Q1, Q1 // 4)

    plsc.subcore_barrier()

    pltpu.sync_copy(acc_s.at[pl.ds(base, TR)], sum_out.at[c, pl.ds(base, TR)])


@functools.partial(
    pl.kernel,
    out_type=jax.ShapeDtypeStruct((NC, NPAD, D), jnp.float32),
    mesh=_mesh,
    scratch_types=[
        pltpu.VMEM((IH, K), jnp.int32),
        pltpu.VMEM((K, D), jnp.float32),
        pltpu.SemaphoreType.DMA,
        pltpu.VMEM_SHARED((NPAD, D), jnp.float32),
    ],
)
def _sc_ones(dst_hbm, cnt_out, didx, ones_v, ssem, acc_s):
    c = lax.axis_index("c")
    s = lax.axis_index("s")
    w = c * NS + s
    base = s * TR

    _fill(ones_v, K, 0.0)
    _zero_slab(ones_v, acc_s, base)
    _fill(ones_v, K, 1.0)

    plsc.subcore_barrier()

    def drain():
        pltpu.make_async_copy(ones_v, acc_s.at[pl.ds(0, K)], ssem).wait()

    def edge_body(i, _):
        pltpu.async_copy(ones_v, acc_s.at[didx.at[i]], ssem, add=True)

        @pl.when(i >= 3)
        def _():
            drain()
        return 0

    for half in range(2):
        pltpu.sync_copy(dst_hbm.at[0, pl.ds(w * CH + half * IH, IH)], didx)
        lax.fori_loop(0, IH, edge_body, 0)
        for _ in range(3):
            drain()

    plsc.subcore_barrier()

    pltpu.sync_copy(acc_s.at[pl.ds(base, TR)], cnt_out.at[c, pl.ds(base, TR)])


_BM = 400
_G = N // _BM


def _dotT(a, b):
    return lax.dot_general(a, b, (((1,), (1,)), ((), ())),
                           preferred_element_type=jnp.float32)


def _tc1_body(p_ref, c_ref, x_ref, wl_ref, wr_ref, b_ref, o_ref):
    cnt = jnp.maximum(c_ref[0, :, 0:1] + c_ref[1, :, 0:1], 1.0)
    mean = (p_ref[0] + p_ref[1]) / cnt
    h = _dotT(mean, wl_ref[...]) + _dotT(x_ref[...], wr_ref[...]) + b_ref[...]
    o_ref[...] = jnp.maximum(h, 0.0)


_tc1 = pl.pallas_call(
    _tc1_body,
    grid=(_G,),
    in_specs=[
        pl.BlockSpec((NC, _BM, D), lambda i: (0, i, 0)),
        pl.BlockSpec((NC, _BM, D), lambda i: (0, i, 0)),
        pl.BlockSpec((_BM, D), lambda i: (i, 0)),
        pl.BlockSpec((D, D), lambda i: (0, 0)),
        pl.BlockSpec((D, D), lambda i: (0, 0)),
        pl.BlockSpec((1, D), lambda i: (0, 0)),
    ],
    out_specs=pl.BlockSpec((_BM, D), lambda i: (i, 0)),
    out_shape=jax.ShapeDtypeStruct((N, D), jnp.float32),
)


def _tc2_body(q_ref, c_ref, h_ref, w2l_ref, w2r_ref, b2_ref,
              wm1_ref, bm1_ref, g2_ref, bt_ref, wm2_ref, bm2_ref, o_ref):
    cnt = jnp.maximum(c_ref[0, :, 0:1] + c_ref[1, :, 0:1], 1.0)
    mean = (q_ref[0] + q_ref[1]) / cnt
    h2 = _dotT(mean, w2l_ref[...]) + _dotT(h_ref[...], w2r_ref[...]) + b2_ref[...]
    h2 = jnp.maximum(h2, 0.0)
    z = (_dotT(h2, wm1_ref[...]) + bm1_ref[...]) * g2_ref[...] + bt_ref[...]
    z = jnp.maximum(z, 0.0)
    o_ref[...] = _dotT(z, wm2_ref[...]) + bm2_ref[...]


_tc2 = pl.pallas_call(
    _tc2_body,
    grid=(_G,),
    in_specs=[
        pl.BlockSpec((NC, _BM, D), lambda i: (0, i, 0)),
        pl.BlockSpec((NC, _BM, D), lambda i: (0, i, 0)),
        pl.BlockSpec((_BM, D), lambda i: (i, 0)),
        pl.BlockSpec((D, D), lambda i: (0, 0)),
        pl.BlockSpec((D, D), lambda i: (0, 0)),
        pl.BlockSpec((1, D), lambda i: (0, 0)),
        pl.BlockSpec((D, D), lambda i: (0, 0)),
        pl.BlockSpec((1, D), lambda i: (0, 0)),
        pl.BlockSpec((1, D), lambda i: (0, 0)),
        pl.BlockSpec((1, D), lambda i: (0, 0)),
        pl.BlockSpec((EMB, D), lambda i: (0, 0)),
        pl.BlockSpec((1, EMB), lambda i: (0, 0)),
    ],
    out_specs=pl.BlockSpec((_BM, EMB), lambda i: (i, 0)),
    out_shape=jax.ShapeDtypeStruct((N, EMB), jnp.float32),
)


def kernel(x, edge_index, W1l, b1l, W1r, W2l, b2l, W2r,
           Wm1, bm1, gamma, beta, Wm2, bm2):
    ei = edge_index.astype(jnp.int32)
    pad = EPAD - E
    src_p = jnp.concatenate([ei[0], jnp.zeros((pad,), jnp.int32)]).reshape(1, TCH, K)
    dst_p = jnp.concatenate([ei[1], jnp.full((pad,), N, jnp.int32)]).reshape(1, TCH, K)

    c1 = _sc_ones(dst_p)
    s1 = _sc_sum(x, src_p, dst_p)
    h = _tc1(s1, c1, x, W1l, W1r, b1l.reshape(1, D))
    s2 = _sc_sum(h, src_p, dst_p)
    g2 = (gamma / jnp.sqrt(1.0 + 1e-5)).reshape(1, D)
    out = _tc2(s2, c1, h, W2l, W2r, b2l.reshape(1, D),
               Wm1, bm1.reshape(1, D), g2, beta.reshape(1, D),
               Wm2, bm2.reshape(1, EMB))
    return out


# speedup vs baseline: 1.1002x; 1.0077x over previous
import functools

import jax
import jax.numpy as jnp
from jax import lax
from jax.experimental import pallas as pl
from jax.experimental.pallas import tpu as pltpu
from jax.experimental.pallas import tpu_sc as plsc

N = 10000
D = 128
EMB = 64
E = 320000
NC, NS = 2, 16
NW = NC * NS
K = 128
EPW = 10240
CH = EPW // K
IH = CH // 2
EPAD = NW * EPW
TCH = EPAD // K
Q0 = 148
Q1 = 12
NPAD = 10112
TR = NPAD // NS

_mesh = plsc.VectorSubcoreMesh(core_axis_name="c", subcore_axis_name="s")


def _fill(ref, nrows, val):
    def body(i, _):
        ref[i // 8, pl.ds((i % 8) * 16, 16)] = jnp.full((16,), val, jnp.float32)
        return 0
    lax.fori_loop(0, nrows * 8, body, 0)


def _zero_slab(zsrc, acc_s, base):
    for j in range(TR // K):
        pltpu.sync_copy(zsrc, acc_s.at[pl.ds(base + j * K, K)])
    rem = TR - (TR // K) * K
    if rem:
        pltpu.sync_copy(zsrc.at[pl.ds(0, rem)],
                        acc_s.at[pl.ds(base + (TR // K) * K, rem)])


@functools.partial(
    pl.kernel,
    out_type=jax.ShapeDtypeStruct((NC, NPAD, D), jnp.float32),
    mesh=_mesh,
    scratch_types=[
        pltpu.VMEM((4, K), jnp.int32),
        pltpu.VMEM((4, K), jnp.int32),
        pltpu.VMEM((K, D), jnp.float32),
        pltpu.VMEM((K, D), jnp.float32),
        pltpu.SemaphoreType.DMA,
        pltpu.SemaphoreType.DMA,
        pltpu.VMEM_SHARED((NPAD, D), jnp.float32),
    ],
)
def _sc_sum(x_hbm, src_hbm, dst_hbm, sum_out,
            sidx, didx, rows0, rows1, g0, g1, acc_s):
    c = lax.axis_index("c")
    s = lax.axis_index("s")
    base = s * TR
    rows = (rows0, rows1)
    gsem = (g0, g1)

    _fill(rows0, K, 0.0)
    _zero_slab(rows0, acc_s, base)

    plsc.subcore_barrier()

    def issue_gather(j, b):
        for h in range(2):
            pltpu.async_copy(x_hbm.at[sidx.at[j, pl.ds(64 * h, 64)]],
                             rows[b].at[pl.ds(64 * h, 64)], gsem[b])

    def run_range(w0, nstages):
        def stage_body(g, _):
            ch0 = w0 + 4 * g
            pltpu.sync_copy(src_hbm.at[0, pl.ds(ch0, 4)], sidx)
            pltpu.sync_copy(dst_hbm.at[0, pl.ds(ch0, 4)], didx)
            issue_gather(0, 0)
            issue_gather(1, 1)
            for j in range(4):
                b = j % 2
                pltpu.make_async_copy(x_hbm.at[pl.ds(0, K)], rows[b],
                                      gsem[b]).wait()
                pltpu.sync_copy(rows[b], acc_s.at[didx.at[j]], add=True)
                if j + 2 < 4:
                    issue_gather(j + 2, b)
            return 0
        lax.fori_loop(0, nstages, stage_body, 0)

    @pl.when(c == 0)
    def _():
        run_range(s * Q0, Q0 // 4)

    @pl.when(c == 1)
    def _():
        run_range(NS * Q0 + s * Q1, Q1 // 4)

    plsc.subcore_barrier()

    pltpu.sync_copy(acc_s.at[pl.ds(base, TR)], sum_out.at[c, pl.ds(base, TR)])


@functools.partial(
    pl.kernel,
    out_type=jax.ShapeDtypeStruct((NC, NPAD, D), jnp.float32),
    mesh=_mesh,
    scratch_types=[
        pltpu.VMEM((IH, K), jnp.int32),
        pltpu.VMEM((K, D), jnp.float32),
        pltpu.SemaphoreType.DMA,
        pltpu.VMEM_SHARED((NPAD, D), jnp.float32),
    ],
)
def _sc_ones(dst_hbm, cnt_out, didx, ones_v, ssem, acc_s):
    c = lax.axis_index("c")
    s = lax.axis_index("s")
    w = c * NS + s
    base = s * TR

    _fill(ones_v, K, 0.0)
    _zero_slab(ones_v, acc_s, base)
    _fill(ones_v, K, 1.0)

    plsc.subcore_barrier()

    def drain():
        pltpu.make_async_copy(ones_v, acc_s.at[pl.ds(0, K)], ssem).wait()

    def edge_body(i, _):
        pltpu.async_copy(ones_v, acc_s.at[didx.at[i]], ssem, add=True)

        @pl.when(i >= 3)
        def _():
            drain()
        return 0

    for half in range(2):
        pltpu.sync_copy(dst_hbm.at[0, pl.ds(w * CH + half * IH, IH)], didx)
        lax.fori_loop(0, IH, edge_body, 0)
        for _ in range(3):
            drain()

    plsc.subcore_barrier()

    pltpu.sync_copy(acc_s.at[pl.ds(base, TR)], cnt_out.at[c, pl.ds(base, TR)])


_BM = 400
_G = N // _BM


def _dotT(a, b):
    return lax.dot_general(a, b, (((1,), (1,)), ((), ())),
                           preferred_element_type=jnp.float32)


def _tc1_body(p_ref, c_ref, x_ref, wl_ref, wr_ref, b_ref, o_ref):
    cnt = jnp.maximum(c_ref[0, :, 0:1] + c_ref[1, :, 0:1], 1.0)
    mean = (p_ref[0] + p_ref[1]) / cnt
    h = _dotT(mean, wl_ref[...]) + _dotT(x_ref[...], wr_ref[...]) + b_ref[...]
    o_ref[...] = jnp.maximum(h, 0.0)


_tc1 = pl.pallas_call(
    _tc1_body,
    grid=(_G,),
    in_specs=[
        pl.BlockSpec((NC, _BM, D), lambda i: (0, i, 0)),
        pl.BlockSpec((NC, _BM, D), lambda i: (0, i, 0)),
        pl.BlockSpec((_BM, D), lambda i: (i, 0)),
        pl.BlockSpec((D, D), lambda i: (0, 0)),
        pl.BlockSpec((D, D), lambda i: (0, 0)),
        pl.BlockSpec((1, D), lambda i: (0, 0)),
    ],
    out_specs=pl.BlockSpec((_BM, D), lambda i: (i, 0)),
    out_shape=jax.ShapeDtypeStruct((N, D), jnp.float32),
)


def _tc2_body(q_ref, c_ref, h_ref, w2l_ref, w2r_ref, b2_ref,
              wm1_ref, bm1_ref, g2_ref, bt_ref, wm2_ref, bm2_ref, o_ref):
    cnt = jnp.maximum(c_ref[0, :, 0:1] + c_ref[1, :, 0:1], 1.0)
    mean = (q_ref[0] + q_ref[1]) / cnt
    h2 = _dotT(mean, w2l_ref[...]) + _dotT(h_ref[...], w2r_ref[...]) + b2_ref[...]
    h2 = jnp.maximum(h2, 0.0)
    z = (_dotT(h2, wm1_ref[...]) + bm1_ref[...]) * g2_ref[...] + bt_ref[...]
    z = jnp.maximum(z, 0.0)
    o_ref[...] = _dotT(z, wm2_ref[...]) + bm2_ref[...]


_tc2 = pl.pallas_call(
    _tc2_body,
    grid=(_G,),
    in_specs=[
        pl.BlockSpec((NC, _BM, D), lambda i: (0, i, 0)),
        pl.BlockSpec((NC, _BM, D), lambda i: (0, i, 0)),
        pl.BlockSpec((_BM, D), lambda i: (i, 0)),
        pl.BlockSpec((D, D), lambda i: (0, 0)),
        pl.BlockSpec((D, D), lambda i: (0, 0)),
        pl.BlockSpec((1, D), lambda i: (0, 0)),
        pl.BlockSpec((D, D), lambda i: (0, 0)),
        pl.BlockSpec((1, D), lambda i: (0, 0)),
        pl.BlockSpec((1, D), lambda i: (0, 0)),
        pl.BlockSpec((1, D), lambda i: (0, 0)),
        pl.BlockSpec((EMB, D), lambda i: (0, 0)),
        pl.BlockSpec((1, EMB), lambda i: (0, 0)),
    ],
    out_specs=pl.BlockSpec((_BM, EMB), lambda i: (i, 0)),
    out_shape=jax.ShapeDtypeStruct((N, EMB), jnp.float32),
)


def kernel(x, edge_index, W1l, b1l, W1r, W2l, b2l, W2r,
           Wm1, bm1, gamma, beta, Wm2, bm2):
    ei = edge_index.astype(jnp.int32)
    pad = EPAD - E
    src_p = jnp.concatenate([ei[0], jnp.zeros((pad,), jnp.int32)]).reshape(1, TCH, K)
    dst_p = jnp.concatenate([ei[1], jnp.full((pad,), N, jnp.int32)]).reshape(1, TCH, K)

    c1 = _sc_ones(dst_p)
    s1 = _sc_sum(x, src_p, dst_p)
    h = _tc1(s1, c1, x, W1l, W1r, b1l.reshape(1, D))
    s2 = _sc_sum(h, src_p, dst_p)
    g2 = (gamma / jnp.sqrt(1.0 + 1e-5)).reshape(1, D)
    out = _tc2(s2, c1, h, W2l, W2r, b2l.reshape(1, D),
               Wm1, bm1.reshape(1, D), g2, beta.reshape(1, D),
               Wm2, bm2.reshape(1, EMB))
    return out
